# baseline (speedup 1.0000x reference)
import functools

import jax
import jax.numpy as jnp
from jax import lax
from jax.experimental import pallas as pl
from jax.experimental.pallas import tpu as pltpu
from jax.experimental.pallas import tpu_sc as plsc

K = 5
_NC = 1
_NS = 16
_NW = _NC * _NS
_CHUNK = 128
_NB = 8
_PANEL = 160


def _matmul_kernel(x_ref, w_ref, o_ref):
    o_ref[...] = jnp.dot(x_ref[...], w_ref[...],
                         preferred_element_type=jnp.float32)


def _matmul(X, Wcat, bm=1000):
    n, fin = X.shape
    fout = Wcat.shape[1]
    return pl.pallas_call(
        _matmul_kernel,
        grid=(n // bm,),
        in_specs=[
            pl.BlockSpec((bm, fin), lambda i: (i, 0)),
            pl.BlockSpec((fin, fout), lambda i: (0, 0)),
        ],
        out_specs=pl.BlockSpec((bm, fout), lambda i: (i, 0)),
        out_shape=jax.ShapeDtypeStruct((n, fout), jnp.float32),
    )(X, Wcat)


@functools.lru_cache(maxsize=None)
def _step_sc(n_acc, f, cpw, n):
    rps = n_acc // _NS
    urs = n // _NS
    has_p = True
    has_bpp = True
    panels = [(o, min(_PANEL, urs - o)) for o in range(0, urs, _PANEL)]
    mesh = plsc.VectorSubcoreMesh(core_axis_name="c", subcore_axis_name="s",
                                  num_cores=_NC)
    assert cpw % _NB == 0

    out_type = jax.ShapeDtypeStruct((_NC * n_acc, f), jnp.float32)

    @functools.partial(
        pl.kernel,
        out_type=out_type,
        mesh=mesh,
        compiler_params=pltpu.CompilerParams(use_tc_tiling_on_sc=False),
        scratch_types=[
            pltpu.VMEM((cpw, _CHUNK), jnp.int32),
            pltpu.VMEM((cpw, _CHUNK), jnp.int32),
            pltpu.VMEM((_NB, _CHUNK, f), jnp.float32),
            pltpu.VMEM((_PANEL, f), jnp.float32),
            pltpu.VMEM((_PANEL, f), jnp.float32),
            pltpu.VMEM((_PANEL, f), jnp.float32),
            pltpu.VMEM((_PANEL, f), jnp.float32),
            pltpu.VMEM((_PANEL, f), jnp.float32),
            pltpu.VMEM((_PANEL, f), jnp.float32),
            pltpu.VMEM_SHARED((n, f), jnp.float32),
            pltpu.VMEM_SHARED((n_acc, f), jnp.float32),
            pltpu.SemaphoreType.DMA,
            pltpu.SemaphoreType.DMA,
        ] + [pltpu.SemaphoreType.DMA] * (2 * _NB),
    )
    def step(*refs):
        it = iter(refs)
        y_hbm = next(it)
        d_hbm = next(it)
        p_hbm = next(it)
        bpp_ref = next(it)
        src_hbm = next(it)
        dst_hbm = next(it)
        p_out = next(it)
        (src_v, dst_v, rows_v, yb, db, pb0, pb1, bppb, ub,
         u_sh, acc_sh, isem, psem) = (next(it) for _ in range(13))
        sems = list(it)
        gsems, ssems = sems[:_NB], sems[_NB:]

        cid = lax.axis_index("c")
        sid = lax.axis_index("s")
        wid = sid * _NC + cid
        base = wid * cpw
        csrc = pltpu.async_copy(src_hbm.at[pl.ds(base, cpw)], src_v, isem)
        cdst = pltpu.async_copy(dst_hbm.at[pl.ds(base, cpw)], dst_v, isem)

        def zrow(r8, _):
            for rr in range(8):
                for q in range(f // 16):
                    ub[r8 * 8 + rr, pl.ds(q * 16, 16)] = (
                        jnp.zeros((16,), jnp.float32))
            return 0
        lax.fori_loop(0, _PANEL // 8, zrow, 0)
        abase = sid * rps
        for zo in range(0, rps, _PANEL):
            zc = min(_PANEL, rps - zo)
            pltpu.sync_copy(ub.at[pl.ds(0, zc)],
                            acc_sh.at[pl.ds(abase + zo, zc)])

        ubase = sid * urs
        for (off, cnt) in panels:
            r0 = ubase + off
            waits = [pltpu.async_copy(
                y_hbm.at[pl.ds(r0, cnt)], yb.at[pl.ds(0, cnt)], psem)]
            waits.append(pltpu.async_copy(
                d_hbm.at[pl.ds(r0, cnt)], db.at[pl.ds(0, cnt)], psem))
            if has_p:
                waits.append(pltpu.async_copy(
                    p_hbm.at[pl.ds(r0, cnt)], pb0.at[pl.ds(0, cnt)], psem))
                if _NC == 2:
                    waits.append(pltpu.async_copy(
                        p_hbm.at[pl.ds(n_acc + r0, cnt)],
                        pb1.at[pl.ds(0, cnt)], psem))
            if has_bpp:
                waits.append(pltpu.async_copy(
                    bpp_ref.at[pl.ds(r0, cnt)], bppb.at[pl.ds(0, cnt)], psem))
            for w in waits:
                w.wait()

            def rowbody(r, _):
                for q in range(f // 16):
                    sl = pl.ds(q * 16, 16)
                    y = yb[r, sl]
                    dv = db[r, sl]
                    if has_p:
                        s = pb0[r, sl]
                        if _NC == 2:
                            s = s + pb1[r, sl]
                        b = y - 2.0 * dv * s
                    else:
                        b = y
                    if has_bpp:
                        b = b - bppb[r, sl]
                    ub[r, sl] = dv * b
                return 0
            lax.fori_loop(0, cnt, rowbody, 0)

            pltpu.sync_copy(ub.at[pl.ds(0, cnt)],
                            u_sh.at[pl.ds(r0, cnt)])

        csrc.wait()
        cdst.wait()
        plsc.subcore_barrier()

        def wait_gather(m):
            pltpu.make_async_copy(u_sh.at[src_v.at[0]],
                                  rows_v.at[m], gsems[m]).wait()

        def wait_scatter(m):
            pltpu.make_async_copy(rows_v.at[m], acc_sh.at[dst_v.at[0]],
                                  ssems[m]).wait()

        for m in range(_NB):
            pltpu.async_copy(u_sh.at[src_v.at[m]], rows_v.at[m], gsems[m])

        def estep(i, _):
            for m in range(_NB):
                wait_gather(m)
                pltpu.async_copy(rows_v.at[m],
                                 acc_sh.at[dst_v.at[i * _NB + m]],
                                 ssems[m], add=True)
            for m in range(_NB):
                wait_scatter(m)
                nxt = (i + 1) * _NB + m
                nxt = jnp.where(nxt >= cpw, nxt - cpw, nxt)
                pltpu.async_copy(u_sh.at[src_v.at[nxt]], rows_v.at[m],
                                 gsems[m])
            return 0
        lax.fori_loop(0, cpw // _NB, estep, 0)
        for m in range(_NB):
            wait_gather(m)

        plsc.subcore_barrier()
        pltpu.sync_copy(acc_sh.at[pl.ds(abase, rps)],
                        p_out.at[pl.ds(cid * n_acc + abase, rps)])

    return step


@functools.lru_cache(maxsize=None)
def _deg_sc(n_acc, f, cpw):
    rps = n_acc // _NS
    mesh = plsc.VectorSubcoreMesh(core_axis_name="c", subcore_axis_name="s",
                                  num_cores=_NC)

    @functools.partial(
        pl.kernel,
        out_type=jax.ShapeDtypeStruct((_NC, n_acc, f), jnp.float32),
        mesh=mesh,
        compiler_params=pltpu.CompilerParams(use_tc_tiling_on_sc=False),
        scratch_types=[
            pltpu.VMEM((cpw, _CHUNK), jnp.int32),
            pltpu.VMEM((_CHUNK, f), jnp.float32),
            pltpu.VMEM((rps, f), jnp.float32),
            pltpu.VMEM_SHARED((n_acc, f), jnp.float32),
        ],
    )
    def deg(dst_hbm, out_hbm, dst_v, ones_v, zero_v, acc_sh):
        cid = lax.axis_index("c")
        sid = lax.axis_index("s")
        wid = sid * _NC + cid
        pltpu.sync_copy(dst_hbm.at[pl.ds(wid * cpw, cpw)], dst_v)

        def orow(r8, _):
            for rr in range(8):
                for q in range(f // 16):
                    ones_v[r8 * 8 + rr, pl.ds(q * 16, 16)] = (
                        jnp.ones((16,), jnp.float32))
            return 0
        lax.fori_loop(0, _CHUNK // 8, orow, 0)

        def zrow(r8, _):
            for rr in range(8):
                for q in range(f // 16):
                    zero_v[r8 * 8 + rr, pl.ds(q * 16, 16)] = (
                        jnp.zeros((16,), jnp.float32))
            return 0
        lax.fori_loop(0, rps // 8, zrow, 0)
        pltpu.sync_copy(zero_v, acc_sh.at[pl.ds(sid * rps, rps)])
        plsc.subcore_barrier()

        def step(j, _):
            pltpu.sync_copy(ones_v, acc_sh.at[dst_v.at[j]], add=True)
            return 0
        lax.fori_loop(0, cpw, step, 0)

        plsc.subcore_barrier()
        pltpu.sync_copy(acc_sh.at[pl.ds(sid * rps, rps)],
                        out_hbm.at[cid, pl.ds(sid * rps, rps)])

    return deg


def kernel(H, edge_index, W0, W1, W2):
    n, f0 = H.shape
    e_num = edge_index.shape[1]

    egrp = _NW * _CHUNK * _NB
    e_pad = ((e_num + egrp - 1) // egrp) * egrp
    n_chunks = e_pad // _CHUNK
    cpw = n_chunks // _NW
    n_acc = ((n + 1 + _NS * 8 - 1) // (_NS * 8)) * (_NS * 8)
    pad = e_pad - e_num
    src = jnp.concatenate([edge_index[0], jnp.zeros((pad,), jnp.int32)])
    dst = jnp.concatenate([edge_index[1], jnp.full((pad,), n, jnp.int32)])
    src2d = src.reshape(n_chunks, _CHUNK)
    dst2d = dst.reshape(n_chunks, _CHUNK)

    degp = _deg_sc(n_acc, 16, cpw)(dst2d)
    deg = sum(degp[c, :n, 0] for c in range(_NC))
    d = lax.rsqrt(jnp.maximum(deg, 1.0))
    dcol = d[:, None]

    def cheb_layer(X, W):
        f = W.shape[2]
        Wcat = jnp.concatenate([W[k] for k in range(K)], axis=1)
        Y = _matmul(X, Wcat)
        dful = jnp.broadcast_to(dcol, (n, f))
        Yk = [Y[:, k * f:(k + 1) * f] for k in range(K)]
        p4 = _step_sc(n_acc, f, cpw, n, 'head')(
            Yk[4], dful, src2d, dst2d)
        p3, b3 = _step_sc(n_acc, f, cpw, n, 'mid3')(
            Yk[3], dful, p4, src2d, dst2d)
        p2, b2 = _step_sc(n_acc, f, cpw, n, 'mid2')(
            Yk[2], dful, p3, Yk[4], src2d, dst2d)
        p1 = _step_sc(n_acc, f, cpw, n, 'mid1')(
            Yk[1], dful, p2, b3, src2d, dst2d)
        return Yk[0] - dcol * (p1[0, :n] + p1[1, :n]) - b2

    def cheb_layer(X, W):
        f = W.shape[2]
        Wcat = jnp.concatenate([W[k] for k in range(K)], axis=1)
        Y = _matmul(X, Wcat)
        dful = jnp.broadcast_to(dcol, (n, f))
        Yk = [Y[:, k * f:(k + 1) * f] for k in range(K)]
        step = _step_sc(n_acc, f, cpw, n)
        zp = jnp.zeros((_NC * n_acc, f), jnp.float32)
        zb = jnp.zeros((n, f), jnp.float32)

        def psum(p):
            s = p[:n]
            for c in range(1, _NC):
                s = s + p[c * n_acc:c * n_acc + n]
            return s

        p4 = step(Yk[4], dful, zp, zb, src2d, dst2d)
        p3 = step(Yk[3], dful, p4, zb, src2d, dst2d)
        b3 = Yk[3] - 2.0 * dcol * psum(p4)
        p2 = step(Yk[2], dful, p3, Yk[4], src2d, dst2d)
        p1 = step(Yk[1], dful, p2, b3, src2d, dst2d)
        b2 = Yk[2] - 2.0 * dcol * psum(p3) - Yk[4]
        return Yk[0] - dcol * psum(p1) - b2

    X = H
    feats = []
    for W in (W0, W1, W2):
        X = cheb_layer(X, W)
        feats.append(X)
        if len(feats) < 3:
            X = jax.nn.relu(X)
    logp = jax.nn.log_softmax(X, axis=1)
    return (logp, feats[0], feats[1], feats[2])


# speedup vs baseline: 1.1594x; 1.1594x over previous
import functools

import jax
import jax.numpy as jnp
from jax import lax
from jax.experimental import pallas as pl
from jax.experimental.pallas import tpu as pltpu
from jax.experimental.pallas import tpu_sc as plsc

K = 5
_NC = 2
_NS = 16
_NW = _NC * _NS
_CHUNK = 128
_NB = 8
_PANEL = 160


def _matmul_kernel(x_ref, w_ref, o_ref):
    o_ref[...] = jnp.dot(x_ref[...], w_ref[...],
                         preferred_element_type=jnp.float32)


def _matmul(X, Wcat, bm=1000):
    n, fin = X.shape
    fout = Wcat.shape[1]
    return pl.pallas_call(
        _matmul_kernel,
        grid=(n // bm,),
        in_specs=[
            pl.BlockSpec((bm, fin), lambda i: (i, 0)),
            pl.BlockSpec((fin, fout), lambda i: (0, 0)),
        ],
        out_specs=pl.BlockSpec((bm, fout), lambda i: (i, 0)),
        out_shape=jax.ShapeDtypeStruct((n, fout), jnp.float32),
    )(X, Wcat)


@functools.lru_cache(maxsize=None)
def _step_sc(n_acc, f, cpw, n, kind):
    rps = n_acc // _NS
    urs = n // _NS
    has_p = kind != 'head'
    has_bpp = kind == 'midb'
    panels = [(o, min(_PANEL, urs - o)) for o in range(0, urs, _PANEL)]
    mesh = plsc.VectorSubcoreMesh(core_axis_name="c", subcore_axis_name="s",
                                  num_cores=_NC)
    assert cpw % _NB == 0

    out_type = jax.ShapeDtypeStruct((_NC * n_acc, f), jnp.float32)

    @functools.partial(
        pl.kernel,
        out_type=out_type,
        mesh=mesh,
        compiler_params=pltpu.CompilerParams(use_tc_tiling_on_sc=False),
        scratch_types=[
            pltpu.VMEM((cpw, _CHUNK), jnp.int32),
            pltpu.VMEM((cpw, _CHUNK), jnp.int32),
            pltpu.VMEM((_NB, _CHUNK, f), jnp.float32),
            pltpu.VMEM((_PANEL, f), jnp.float32),
            pltpu.VMEM((_PANEL, f), jnp.float32),
            pltpu.VMEM((_PANEL, f), jnp.float32),
            pltpu.VMEM((_PANEL, f), jnp.float32),
            pltpu.VMEM((_PANEL, f), jnp.float32),
            pltpu.VMEM((_PANEL, f), jnp.float32),
            pltpu.VMEM_SHARED((n, f), jnp.float32),
            pltpu.VMEM_SHARED((n_acc, f), jnp.float32),
            pltpu.SemaphoreType.DMA,
            pltpu.SemaphoreType.DMA,
        ] + [pltpu.SemaphoreType.DMA] * (2 * _NB),
    )
    def step(*refs):
        it = iter(refs)
        y_hbm = next(it)
        d_hbm = next(it)
        p_hbm = next(it) if has_p else None
        bpp_ref = next(it) if has_bpp else None
        src_hbm = next(it)
        dst_hbm = next(it)
        p_out = next(it)
        (src_v, dst_v, rows_v, yb, db, pb0, pb1, bppb, ub,
         u_sh, acc_sh, isem, psem) = (next(it) for _ in range(13))
        sems = list(it)
        gsems, ssems = sems[:_NB], sems[_NB:]

        cid = lax.axis_index("c")
        sid = lax.axis_index("s")
        wid = sid * _NC + cid
        base = wid * cpw
        csrc = pltpu.async_copy(src_hbm.at[pl.ds(base, cpw)], src_v, isem)
        cdst = pltpu.async_copy(dst_hbm.at[pl.ds(base, cpw)], dst_v, isem)

        def zrow(r8, _):
            for rr in range(8):
                for q in range(f // 16):
                    ub[r8 * 8 + rr, pl.ds(q * 16, 16)] = (
                        jnp.zeros((16,), jnp.float32))
            return 0
        lax.fori_loop(0, _PANEL // 8, zrow, 0)
        abase = sid * rps
        for zo in range(0, rps, _PANEL):
            zc = min(_PANEL, rps - zo)
            pltpu.sync_copy(ub.at[pl.ds(0, zc)],
                            acc_sh.at[pl.ds(abase + zo, zc)])

        ubase = sid * urs
        for (off, cnt) in panels:
            r0 = ubase + off
            waits = [pltpu.async_copy(
                y_hbm.at[pl.ds(r0, cnt)], yb.at[pl.ds(0, cnt)], psem)]
            waits.append(pltpu.async_copy(
                d_hbm.at[pl.ds(r0, cnt)], db.at[pl.ds(0, cnt)], psem))
            if has_p:
                waits.append(pltpu.async_copy(
                    p_hbm.at[pl.ds(r0, cnt)], pb0.at[pl.ds(0, cnt)], psem))
                if _NC == 2:
                    waits.append(pltpu.async_copy(
                        p_hbm.at[pl.ds(n_acc + r0, cnt)],
                        pb1.at[pl.ds(0, cnt)], psem))
            if has_bpp:
                waits.append(pltpu.async_copy(
                    bpp_ref.at[pl.ds(r0, cnt)], bppb.at[pl.ds(0, cnt)], psem))
            for w in waits:
                w.wait()

            def rowbody(r, _):
                for q in range(f // 16):
                    sl = pl.ds(q * 16, 16)
                    y = yb[r, sl]
                    dv = db[r, sl]
                    if has_p:
                        s = pb0[r, sl]
                        if _NC == 2:
                            s = s + pb1[r, sl]
                        b = y - 2.0 * dv * s
                    else:
                        b = y
                    if has_bpp:
                        b = b - bppb[r, sl]
                    ub[r, sl] = dv * b
                return 0
            lax.fori_loop(0, cnt, rowbody, 0)

            pltpu.sync_copy(ub.at[pl.ds(0, cnt)],
                            u_sh.at[pl.ds(r0, cnt)])

        csrc.wait()
        cdst.wait()
        plsc.subcore_barrier()

        def wait_gather(m):
            pltpu.make_async_copy(u_sh.at[src_v.at[0]],
                                  rows_v.at[m], gsems[m]).wait()

        def wait_scatter(m):
            pltpu.make_async_copy(rows_v.at[m], acc_sh.at[dst_v.at[0]],
                                  ssems[m]).wait()

        for m in range(_NB):
            pltpu.async_copy(u_sh.at[src_v.at[m]], rows_v.at[m], gsems[m])

        def estep(i, _):
            for m in range(_NB):
                wait_gather(m)
                pltpu.async_copy(rows_v.at[m],
                                 acc_sh.at[dst_v.at[i * _NB + m]],
                                 ssems[m], add=True)
            for m in range(_NB):
                wait_scatter(m)
                nxt = (i + 1) * _NB + m
                nxt = jnp.where(nxt >= cpw, nxt - cpw, nxt)
                pltpu.async_copy(u_sh.at[src_v.at[nxt]], rows_v.at[m],
                                 gsems[m])
            return 0
        lax.fori_loop(0, cpw // _NB, estep, 0)
        for m in range(_NB):
            wait_gather(m)

        plsc.subcore_barrier()
        pltpu.sync_copy(acc_sh.at[pl.ds(abase, rps)],
                        p_out.at[pl.ds(cid * n_acc + abase, rps)])

    return step


@functools.lru_cache(maxsize=None)
def _deg_sc(n_acc, f, cpw):
    rps = n_acc // _NS
    mesh = plsc.VectorSubcoreMesh(core_axis_name="c", subcore_axis_name="s",
                                  num_cores=_NC)

    @functools.partial(
        pl.kernel,
        out_type=jax.ShapeDtypeStruct((_NC, n_acc, f), jnp.float32),
        mesh=mesh,
        compiler_params=pltpu.CompilerParams(use_tc_tiling_on_sc=False),
        scratch_types=[
            pltpu.VMEM((cpw, _CHUNK), jnp.int32),
            pltpu.VMEM((_CHUNK, f), jnp.float32),
            pltpu.VMEM((rps, f), jnp.float32),
            pltpu.VMEM_SHARED((n_acc, f), jnp.float32),
        ],
    )
    def deg(dst_hbm, out_hbm, dst_v, ones_v, zero_v, acc_sh):
        cid = lax.axis_index("c")
        sid = lax.axis_index("s")
        wid = sid * _NC + cid
        pltpu.sync_copy(dst_hbm.at[pl.ds(wid * cpw, cpw)], dst_v)

        def orow(r8, _):
            for rr in range(8):
                for q in range(f // 16):
                    ones_v[r8 * 8 + rr, pl.ds(q * 16, 16)] = (
                        jnp.ones((16,), jnp.float32))
            return 0
        lax.fori_loop(0, _CHUNK // 8, orow, 0)

        def zrow(r8, _):
            for rr in range(8):
                for q in range(f // 16):
                    zero_v[r8 * 8 + rr, pl.ds(q * 16, 16)] = (
                        jnp.zeros((16,), jnp.float32))
            return 0
        lax.fori_loop(0, rps // 8, zrow, 0)
        pltpu.sync_copy(zero_v, acc_sh.at[pl.ds(sid * rps, rps)])
        plsc.subcore_barrier()

        def step(j, _):
            pltpu.sync_copy(ones_v, acc_sh.at[dst_v.at[j]], add=True)
            return 0
        lax.fori_loop(0, cpw, step, 0)

        plsc.subcore_barrier()
        pltpu.sync_copy(acc_sh.at[pl.ds(sid * rps, rps)],
                        out_hbm.at[cid, pl.ds(sid * rps, rps)])

    return deg


def kernel(H, edge_index, W0, W1, W2):
    n, f0 = H.shape
    e_num = edge_index.shape[1]

    egrp = _NW * _CHUNK * _NB
    e_pad = ((e_num + egrp - 1) // egrp) * egrp
    n_chunks = e_pad // _CHUNK
    cpw = n_chunks // _NW
    n_acc = ((n + 1 + _NS * 8 - 1) // (_NS * 8)) * (_NS * 8)
    pad = e_pad - e_num
    src = jnp.concatenate([edge_index[0], jnp.zeros((pad,), jnp.int32)])
    dst = jnp.concatenate([edge_index[1], jnp.full((pad,), n, jnp.int32)])
    src2d = src.reshape(n_chunks, _CHUNK)
    dst2d = dst.reshape(n_chunks, _CHUNK)

    degp = _deg_sc(n_acc, 16, cpw)(dst2d)
    deg = sum(degp[c, :n, 0] for c in range(_NC))
    d = lax.rsqrt(jnp.maximum(deg, 1.0))
    dcol = d[:, None]

    def cheb_layer(X, W):
        f = W.shape[2]
        Wcat = jnp.concatenate([W[k] for k in range(K)], axis=1)
        Y = _matmul(X, Wcat)
        dful = jnp.broadcast_to(dcol, (n, f))
        Yk = [Y[:, k * f:(k + 1) * f] for k in range(K)]
        p4 = _step_sc(n_acc, f, cpw, n, 'head')(
            Yk[4], dful, src2d, dst2d)
        p3, b3 = _step_sc(n_acc, f, cpw, n, 'mid3')(
            Yk[3], dful, p4, src2d, dst2d)
        p2, b2 = _step_sc(n_acc, f, cpw, n, 'mid2')(
            Yk[2], dful, p3, Yk[4], src2d, dst2d)
        p1 = _step_sc(n_acc, f, cpw, n, 'mid1')(
            Yk[1], dful, p2, b3, src2d, dst2d)
        return Yk[0] - dcol * (p1[0, :n] + p1[1, :n]) - b2

    def cheb_layer(X, W):
        f = W.shape[2]
        Wcat = jnp.concatenate([W[k] for k in range(K)], axis=1)
        Y = _matmul(X, Wcat)
        dful = jnp.broadcast_to(dcol, (n, f))
        Yk = [Y[:, k * f:(k + 1) * f] for k in range(K)]
        def psum(p):
            s = p[:n]
            for c in range(1, _NC):
                s = s + p[c * n_acc:c * n_acc + n]
            return s

        mk = _step_sc
        p4 = mk(n_acc, f, cpw, n, 'head')(Yk[4], dful, src2d, dst2d)
        p3 = mk(n_acc, f, cpw, n, 'mid')(Yk[3], dful, p4, src2d, dst2d)
        b3 = Yk[3] - 2.0 * dcol * psum(p4)
        p2 = mk(n_acc, f, cpw, n, 'midb')(Yk[2], dful, p3, Yk[4], src2d, dst2d)
        p1 = mk(n_acc, f, cpw, n, 'midb')(Yk[1], dful, p2, b3, src2d, dst2d)
        b2 = Yk[2] - 2.0 * dcol * psum(p3) - Yk[4]
        return Yk[0] - dcol * psum(p1) - b2

    X = H
    feats = []
    for W in (W0, W1, W2):
        X = cheb_layer(X, W)
        feats.append(X)
        if len(feats) < 3:
            X = jax.nn.relu(X)
    logp = jax.nn.log_softmax(X, axis=1)
    return (logp, feats[0], feats[1], feats[2])
